# Initial kernel scaffold; baseline (speedup 1.0000x reference)
import jax
import jax.numpy as jnp
from jax.experimental import pallas as pl


def kernel(x, edge_index, batch, eig, stats, W1, b1, g1, be1, W2, b2, W3, b3, g2, be2, W4, b4, LW1, Lb1, LW2, Lb2):
    raise NotImplementedError("write your pallas kernel here")


# speedup vs baseline: 6.2360x; 6.2360x over previous
import functools

import jax
import jax.numpy as jnp
from jax import lax
from jax.experimental import pallas as pl
from jax.experimental.pallas import tpu as pltpu
from jax.experimental.pallas import tpu_sc as plsc

N = 50000
E = 800000
B = 8
DH = 64
HALF = 32
QW = 16
NC = 2
NS = 16

E_PAD = 819200
N_ACC = 50048
RPT = N_ACC // NS
CHUNK = 1024
IDXW = 128
ROWBLK = 2000
NSTEP = N // ROWBLK

def _agg1_body(x_hbm, src_hbm, dst_hbm, zeros_hbm, out_hbm,
               src_v, dst_v, rows_v, accum, sem):
    c = lax.axis_index("c")
    s = lax.axis_index("s")
    w = c * NS + s
    pltpu.sync_copy(zeros_hbm, accum.at[pl.ds(s * RPT, RPT)])
    plsc.subcore_barrier()
    row_base = w * (E_PAD // 32 // IDXW)

    def body(i, carry):
        r = row_base + i * 8
        pltpu.sync_copy(src_hbm.at[pl.ds(r, 8)], src_v)
        pltpu.sync_copy(dst_hbm.at[pl.ds(r, 8)], dst_v)
        descs = [
            pltpu.async_copy(x_hbm.at[src_v.at[j]],
                             rows_v.at[pl.ds(j * IDXW, IDXW)], sem)
            for j in range(8)
        ]
        for d in descs:
            d.wait()
        for j in range(8):
            pltpu.sync_copy(rows_v.at[pl.ds(j * IDXW, IDXW)],
                            accum.at[dst_v.at[j]], add=True)
        return carry

    lax.fori_loop(0, E_PAD // 32 // CHUNK, body, 0)
    plsc.subcore_barrier()
    pltpu.sync_copy(accum.at[pl.ds(s * RPT, RPT)],
                    out_hbm.at[c, pl.ds(s * RPT, RPT)])


def _agg2_body(hq_hbm, src_hbm, dst_hbm, zeros_hbm, out_hbm,
               src_v, dst_v, rows_v, accum, sem):
    c = lax.axis_index("c")
    s = lax.axis_index("s")
    row_base = s * (E_PAD // 16 // IDXW)

    for q in range(2):
        qi = c * 2 + q
        pltpu.sync_copy(zeros_hbm, accum.at[pl.ds(s * RPT, RPT)])
        plsc.subcore_barrier()

        def body(i, carry):
            r = row_base + i * 8
            pltpu.sync_copy(src_hbm.at[pl.ds(r, 8)], src_v)
            pltpu.sync_copy(dst_hbm.at[pl.ds(r, 8)], dst_v)
            descs = [
                pltpu.async_copy(hq_hbm.at[qi].at[src_v.at[j]],
                                 rows_v.at[pl.ds(j * IDXW, IDXW)], sem)
                for j in range(8)
            ]
            for d in descs:
                d.wait()
            for j in range(8):
                pltpu.sync_copy(rows_v.at[pl.ds(j * IDXW, IDXW)],
                                accum.at[dst_v.at[j]], add=True)
            return carry

        lax.fori_loop(0, E_PAD // 16 // CHUNK, body, 0)
        plsc.subcore_barrier()
        pltpu.sync_copy(accum.at[pl.ds(s * RPT, RPT)],
                        out_hbm.at[qi, pl.ds(s * RPT, RPT)])


@functools.lru_cache(maxsize=None)
def _sc_aggs():
    mesh = plsc.VectorSubcoreMesh(core_axis_name="c", subcore_axis_name="s",
                                  num_cores=NC, num_subcores=NS)
    params = pltpu.CompilerParams(use_tc_tiling_on_sc=False)
    agg1 = pl.kernel(
        _agg1_body,
        out_type=jax.ShapeDtypeStruct((NC, N_ACC, QW), jnp.float32),
        mesh=mesh,
        compiler_params=params,
        scratch_types=[
            pltpu.VMEM((8, IDXW), jnp.int32),
            pltpu.VMEM((8, IDXW), jnp.int32),
            pltpu.VMEM((CHUNK, QW), jnp.float32),
            pltpu.VMEM_SHARED((N_ACC, QW), jnp.float32),
            pltpu.SemaphoreType.DMA,
        ],
    )
    agg2 = pl.kernel(
        _agg2_body,
        out_type=jax.ShapeDtypeStruct((4, N_ACC, QW), jnp.float32),
        mesh=mesh,
        compiler_params=params,
        scratch_types=[
            pltpu.VMEM((8, IDXW), jnp.int32),
            pltpu.VMEM((8, IDXW), jnp.int32),
            pltpu.VMEM((CHUNK, QW), jnp.float32),
            pltpu.VMEM_SHARED((N_ACC, QW), jnp.float32),
            pltpu.SemaphoreType.DMA,
        ],
    )
    return agg1, agg2


def _b1_body(x_ref, p0_ref, p1_ref, W1_ref, b1_ref, g1_ref, be1_ref,
             z_ref, s1_ref, t1_ref, ssum_ref, gram_ref, m0_ref):
    i = pl.program_id(0)
    z = x_ref[...] + p0_ref[0][:, :2] + p1_ref[0][:, :2]
    z_ref[...] = z

    @pl.when(i == 0)
    def _():
        m0_ref[...] = jnp.sum(z, axis=0, keepdims=True) / ROWBLK
        ssum_ref[...] = jnp.zeros_like(ssum_ref)
        gram_ref[...] = jnp.zeros_like(gram_ref)

    zc = z - m0_ref[...]
    ssum_ref[...] += jnp.sum(zc, axis=0, keepdims=True)
    gram_ref[...] += lax.dot_general(zc, zc, (((0,), (0,)), ((), ())))

    @pl.when(i == NSTEP - 1)
    def _():
        meanc = ssum_ref[...] / N
        C = gram_ref[...] / N - lax.dot_general(
            meanc, meanc, (((0,), (0,)), ((), ())))
        mean = meanc + m0_ref[...]
        W1 = W1_ref[...].astype(jnp.bfloat16).astype(jnp.float32)
        mu = mean @ W1 + b1_ref[...]
        var = jnp.sum((C @ W1) * W1, axis=0, keepdims=True)
        s = g1_ref[...] * lax.rsqrt(var + 1e-5)
        s1_ref[...] = s
        t1_ref[...] = be1_ref[...] - mu * s


def _bfdot(a, b):
    return jnp.dot(a.astype(jnp.bfloat16), b.astype(jnp.bfloat16),
                   preferred_element_type=jnp.float32)


def _b2_body(z_ref, batch_ref, W1_ref, b1_ref, s1_ref, t1_ref, W2_ref,
             b2_ref, hq_ref, h1p_ref, acc_ref):
    i = pl.program_id(0)
    z = z_ref[...]
    hp = _bfdot(z, W1_ref[...]) + b1_ref[...]
    h = jnp.maximum(hp * s1_ref[...] + t1_ref[...], 0.0)
    h1 = jnp.maximum(_bfdot(h, W2_ref[...]) + b2_ref[...], 0.0)
    hq_ref[0] = h1[:, 0 * QW:1 * QW]
    hq_ref[1] = h1[:, 1 * QW:2 * QW]
    hq_ref[2] = h1[:, 2 * QW:3 * QW]
    hq_ref[3] = h1[:, 3 * QW:4 * QW]
    bt = batch_ref[0]
    oh = (lax.broadcasted_iota(jnp.int32, (B, ROWBLK), 0) == bt
          ).astype(jnp.float32)
    pooled = lax.dot_general(oh, h1, (((1,), (0,)), ((), ())))

    @pl.when(i == 0)
    def _():
        acc_ref[...] = jnp.zeros_like(acc_ref)

    acc_ref[...] += pooled

    @pl.when(i == NSTEP - 1)
    def _():
        h1p_ref[...] = acc_ref[...]


def _d1_body(q0_ref, q1_ref, q2_ref, q3_ref, a0_ref, a1_ref, a2_ref, a3_ref,
             W3_ref, b3_ref, g2_ref, be2_ref,
             u_ref, s2_ref, t2_ref, ssum_ref, gram_ref, m0_ref):
    i = pl.program_id(0)
    u = jnp.concatenate(
        [q0_ref[0] + a0_ref[0], q1_ref[0] + a1_ref[0],
         q2_ref[0] + a2_ref[0], q3_ref[0] + a3_ref[0]], axis=1)
    u_ref[...] = u

    @pl.when(i == 0)
    def _():
        m0_ref[...] = jnp.sum(u, axis=0, keepdims=True) / ROWBLK
        ssum_ref[...] = jnp.zeros_like(ssum_ref)
        gram_ref[...] = jnp.zeros_like(gram_ref)

    uc = u - m0_ref[...]
    ssum_ref[...] += jnp.sum(uc, axis=0, keepdims=True)
    gram_ref[...] += lax.dot_general(uc, uc, (((0,), (0,)), ((), ())))

    @pl.when(i == NSTEP - 1)
    def _():
        meanc = ssum_ref[...] / N
        C = gram_ref[...] / N - lax.dot_general(
            meanc, meanc, (((0,), (0,)), ((), ())))
        mean = meanc + m0_ref[...]
        W3 = W3_ref[...].astype(jnp.bfloat16).astype(jnp.float32)
        mu = mean @ W3 + b3_ref[...]
        var = jnp.sum((C @ W3) * W3, axis=0, keepdims=True)
        s = g2_ref[...] * lax.rsqrt(var + 1e-5)
        s2_ref[...] = s
        t2_ref[...] = be2_ref[...] - mu * s


def _d2_body(u_ref, batch_ref, W3_ref, b3_ref, s2_ref, t2_ref, W4_ref,
             b4_ref, h2p_ref, acc_ref):
    i = pl.program_id(0)
    u = u_ref[...]
    hp = _bfdot(u, W3_ref[...]) + b3_ref[...]
    h = jnp.maximum(hp * s2_ref[...] + t2_ref[...], 0.0)
    h2 = jnp.maximum(_bfdot(h, W4_ref[...]) + b4_ref[...], 0.0)
    bt = batch_ref[0]
    oh = (lax.broadcasted_iota(jnp.int32, (B, ROWBLK), 0) == bt
          ).astype(jnp.float32)
    pooled = lax.dot_general(oh, h2, (((1,), (0,)), ((), ())))

    @pl.when(i == 0)
    def _():
        acc_ref[...] = jnp.zeros_like(acc_ref)

    acc_ref[...] += pooled

    @pl.when(i == NSTEP - 1)
    def _():
        h2p_ref[...] = acc_ref[...]


def _head_body(h1p_ref, h2p_ref, st_ref, LA_ref, LB_ref, LC_ref, Lb1_ref,
               LW2_ref, Lb2_ref, out_ref):
    h = (_bfdot(h1p_ref[...], LA_ref[...]) + _bfdot(h2p_ref[...], LB_ref[...])
         + _bfdot(st_ref[...], LC_ref[...]) + Lb1_ref[...])
    h = jnp.maximum(h, 0.0)
    o = _bfdot(h, LW2_ref[...]) + Lb2_ref[...]
    m = jnp.max(o, axis=1, keepdims=True)
    lse = jnp.log(jnp.sum(jnp.exp(o - m), axis=1, keepdims=True)) + m
    out_ref[...] = o - lse


def _row_spec(cols):
    return pl.BlockSpec((ROWBLK, cols), lambda i: (i, 0))


def _full2(r, c):
    return pl.BlockSpec((r, c), lambda i: (0, 0))


def kernel(x, edge_index, batch, eig, stats, W1, b1, g1, be1, W2, b2,
           W3, b3, g2, be2, W4, b4, LW1, Lb1, LW2, Lb2):
    f32 = jnp.float32
    src = edge_index[0]
    dst = edge_index[1]
    pad = E_PAD - E
    srcp = jnp.concatenate(
        [src, jnp.zeros((pad,), jnp.int32)]).reshape(E_PAD // IDXW, IDXW)
    dstp = jnp.concatenate(
        [dst, jnp.full((pad,), N, jnp.int32)]).reshape(E_PAD // IDXW, IDXW)
    zq = jnp.zeros((RPT, QW), f32)
    x16 = jnp.concatenate([x, jnp.zeros((N, QW - 2), f32)], axis=1)

    sc_agg1, sc_agg2 = _sc_aggs()
    agg1 = sc_agg1(x16, srcp, dstp, zq)

    z_arr, s1, t1 = pl.pallas_call(
        _b1_body,
        grid=(NSTEP,),
        in_specs=[
            _row_spec(2),
            pl.BlockSpec((1, ROWBLK, QW), lambda i: (0, i, 0)),
            pl.BlockSpec((1, ROWBLK, QW), lambda i: (1, i, 0)),
            _full2(2, DH), _full2(1, DH), _full2(1, DH), _full2(1, DH),
        ],
        out_specs=[_row_spec(2), _full2(1, DH), _full2(1, DH)],
        out_shape=[
            jax.ShapeDtypeStruct((N, 2), f32),
            jax.ShapeDtypeStruct((1, DH), f32),
            jax.ShapeDtypeStruct((1, DH), f32),
        ],
        scratch_shapes=[pltpu.VMEM((1, 2), f32), pltpu.VMEM((2, 2), f32),
                        pltpu.VMEM((1, 2), f32)],
    )(x, agg1, agg1, W1, b1.reshape(1, DH), g1.reshape(1, DH),
      be1.reshape(1, DH))

    batch3 = batch.reshape(NSTEP, 1, ROWBLK)
    hq, h1p = pl.pallas_call(
        _b2_body,
        grid=(NSTEP,),
        in_specs=[
            _row_spec(2),
            pl.BlockSpec((1, 1, ROWBLK), lambda i: (i, 0, 0)),
            _full2(2, DH), _full2(1, DH), _full2(1, DH), _full2(1, DH),
            _full2(DH, DH), _full2(1, DH),
        ],
        out_specs=[pl.BlockSpec((4, ROWBLK, QW), lambda i: (0, i, 0)),
                   _full2(B, DH)],
        out_shape=[jax.ShapeDtypeStruct((4, N, QW), f32),
                   jax.ShapeDtypeStruct((B, DH), f32)],
        scratch_shapes=[pltpu.VMEM((B, DH), f32)],
    )(z_arr, batch3, W1, b1.reshape(1, DH), s1, t1, W2, b2.reshape(1, DH))

    aq = sc_agg2(hq, srcp, dstp, zq)

    def _qspec(qi):
        return pl.BlockSpec((1, ROWBLK, QW), lambda i, qi=qi: (qi, i, 0))

    u, s2, t2 = pl.pallas_call(
        _d1_body,
        grid=(NSTEP,),
        in_specs=[_qspec(0), _qspec(1), _qspec(2), _qspec(3),
                  _qspec(0), _qspec(1), _qspec(2), _qspec(3),
                  _full2(DH, DH), _full2(1, DH), _full2(1, DH),
                  _full2(1, DH)],
        out_specs=[_row_spec(DH), _full2(1, DH), _full2(1, DH)],
        out_shape=[
            jax.ShapeDtypeStruct((N, DH), f32),
            jax.ShapeDtypeStruct((1, DH), f32),
            jax.ShapeDtypeStruct((1, DH), f32),
        ],
        scratch_shapes=[pltpu.VMEM((1, DH), f32), pltpu.VMEM((DH, DH), f32),
                        pltpu.VMEM((1, DH), f32)],
    )(hq, hq, hq, hq, aq, aq, aq, aq, W3,
      b3.reshape(1, DH), g2.reshape(1, DH), be2.reshape(1, DH))

    h2p = pl.pallas_call(
        _d2_body,
        grid=(NSTEP,),
        in_specs=[
            _row_spec(DH),
            pl.BlockSpec((1, 1, ROWBLK), lambda i: (i, 0, 0)),
            _full2(DH, DH), _full2(1, DH), _full2(1, DH), _full2(1, DH),
            _full2(DH, DH), _full2(1, DH),
        ],
        out_specs=_full2(B, DH),
        out_shape=jax.ShapeDtypeStruct((B, DH), f32),
        scratch_shapes=[pltpu.VMEM((B, DH), f32)],
    )(u, batch3, W3, b3.reshape(1, DH), s2, t2, W4, b4.reshape(1, DH))

    out = pl.pallas_call(
        _head_body,
        out_shape=jax.ShapeDtypeStruct((B, 5), f32),
    )(h1p, h2p, stats, LW1[:DH], LW1[DH:2 * DH], LW1[2 * DH:],
      Lb1.reshape(1, -1), LW2, Lb2.reshape(1, -1))
    return out
